# Initial kernel scaffold; baseline (speedup 1.0000x reference)
import jax
import jax.numpy as jnp
from jax.experimental import pallas as pl


def kernel(encoder_input_ids, encoder_attention_mask, embedding_table):
    raise NotImplementedError("write your pallas kernel here")


# speedup vs baseline: 1.4514x; 1.4514x over previous
import functools

import jax
import jax.numpy as jnp
from jax import lax
from jax.experimental import pallas as pl
from jax.experimental.pallas import tpu as pltpu
from jax.experimental.pallas import tpu_sc as plsc

_B = 4
_SEQ = 4096
_D = 1024
_HEADS = 16

_NC = 2
_NS = 16
_NW = _NC * _NS
_N_IDS = _B * _SEQ
_PER_W = _N_IDS // _NW
_CHUNK = 32
_N_CHUNKS = _PER_W // _CHUNK


def _gather_body(idx_hbm, table_hbm, out_hbm, idx_v, rows_v, sem):
    wid = lax.axis_index("s") * _NC + lax.axis_index("c")
    base = wid * _PER_W
    pltpu.sync_copy(idx_hbm.at[wid], idx_v)

    def chunk(i, _):
        pltpu.async_copy(table_hbm.at[idx_v.at[i]], rows_v, sem).wait()
        pltpu.sync_copy(rows_v, out_hbm.at[pl.ds(base + i * _CHUNK, _CHUNK)])
        return 0

    lax.fori_loop(0, _N_CHUNKS, chunk, 0)


@jax.jit
def _sc_gather(ids_3d, table):
    mesh = plsc.VectorSubcoreMesh(core_axis_name="c", subcore_axis_name="s")
    f = functools.partial(
        pl.kernel,
        mesh=mesh,
        out_type=jax.ShapeDtypeStruct((_N_IDS, _D), jnp.float32),
        scratch_types=[
            pltpu.VMEM((_N_CHUNKS, _CHUNK), jnp.int32),
            pltpu.VMEM((_CHUNK, _D), jnp.float32),
            pltpu.SemaphoreType.DMA,
        ],
    )(_gather_body)
    return f(ids_3d, table)


def _mask_body(mask_ref, ext_ref, bias_ref):
    ext_ref[...] = (1.0 - mask_ref[...]) * -10000.0
    bias_ref[...] = jnp.zeros_like(bias_ref)


@jax.jit
def _tc_mask(mask):
    return pl.pallas_call(
        _mask_body,
        out_shape=[
            jax.ShapeDtypeStruct((_B, _SEQ), jnp.float32),
            jax.ShapeDtypeStruct((_B * _HEADS, _SEQ), jnp.float32),
        ],
    )(mask)


def kernel(encoder_input_ids, encoder_attention_mask, embedding_table):
    ids = encoder_input_ids.astype(jnp.int32).reshape(_NW, _N_CHUNKS, _CHUNK)
    hidden = _sc_gather(ids, embedding_table).reshape(_B, _SEQ, _D)
    ext, bias = _tc_mask(encoder_attention_mask)
    ext = ext.reshape(_B, 1, 1, _SEQ)
    bias = bias.reshape(_B, _HEADS, _SEQ, 1)
    return (hidden, ext, bias)


# speedup vs baseline: 1.6681x; 1.1493x over previous
import functools

import jax
import jax.numpy as jnp
from jax import lax
from jax.experimental import pallas as pl
from jax.experimental.pallas import tpu as pltpu
from jax.experimental.pallas import tpu_sc as plsc

_B = 4
_SEQ = 4096
_D = 1024
_HEADS = 16

_NC = 2
_NS = 16
_NW = _NC * _NS
_N_IDS = _B * _SEQ
_PER_W = _N_IDS // _NW
_CHUNK = 32
_N_CHUNKS = _PER_W // _CHUNK


def _gather_body(idx_hbm, table_hbm, out_hbm, idx_v, rows0, rows1, gs0, gs1,
                 os0, os1):
    wid = lax.axis_index("s") * _NC + lax.axis_index("c")
    base = wid * _PER_W
    pltpu.sync_copy(idx_hbm.at[wid], idx_v)

    bufs = (rows0, rows1)
    gsems = (gs0, gs1)
    osems = (os0, os1)

    def gather(i, b):
        pltpu.make_async_copy(table_hbm.at[idx_v.at[i]], bufs[b],
                              gsems[b]).start()

    def gather_wait(i, b):
        pltpu.make_async_copy(table_hbm.at[idx_v.at[i]], bufs[b],
                              gsems[b]).wait()

    def out_start(i, b):
        pltpu.make_async_copy(bufs[b],
                              out_hbm.at[pl.ds(base + i * _CHUNK, _CHUNK)],
                              osems[b]).start()

    def out_wait(i, b):
        pltpu.make_async_copy(bufs[b],
                              out_hbm.at[pl.ds(base + i * _CHUNK, _CHUNK)],
                              osems[b]).wait()

    gather(0, 0)
    gather(1, 1)

    def steady(j, _):
        for b in range(2):
            i = 2 * j + b
            gather_wait(i, b)
            out_start(i, b)
            out_wait(i, b)
            gather(i + 2, b)
        return 0

    lax.fori_loop(0, _N_CHUNKS // 2 - 1, steady, 0)

    for b in range(2):
        i = _N_CHUNKS - 2 + b
        gather_wait(i, b)
        out_start(i, b)
    for b in range(2):
        out_wait(_N_CHUNKS - 2 + b, b)


@jax.jit
def _sc_gather(ids_3d, table):
    mesh = plsc.VectorSubcoreMesh(core_axis_name="c", subcore_axis_name="s")
    f = functools.partial(
        pl.kernel,
        mesh=mesh,
        out_type=jax.ShapeDtypeStruct((_N_IDS, _D), jnp.float32),
        scratch_types=[
            pltpu.VMEM((_N_CHUNKS, _CHUNK), jnp.int32),
            pltpu.VMEM((_CHUNK, _D), jnp.float32),
            pltpu.VMEM((_CHUNK, _D), jnp.float32),
            pltpu.SemaphoreType.DMA,
            pltpu.SemaphoreType.DMA,
            pltpu.SemaphoreType.DMA,
            pltpu.SemaphoreType.DMA,
        ],
    )(_gather_body)
    return f(ids_3d, table)


def _mask_body(mask_ref, ext_ref, bias_ref):
    ext_ref[...] = (1.0 - mask_ref[...]) * -10000.0
    bias_ref[...] = jnp.zeros_like(bias_ref)


@jax.jit
def _tc_mask(mask):
    return pl.pallas_call(
        _mask_body,
        out_shape=[
            jax.ShapeDtypeStruct((_B, _SEQ), jnp.float32),
            jax.ShapeDtypeStruct((_B * _HEADS, _SEQ), jnp.float32),
        ],
    )(mask)


def kernel(encoder_input_ids, encoder_attention_mask, embedding_table):
    ids = encoder_input_ids.astype(jnp.int32).reshape(_NW, _N_CHUNKS, _CHUNK)
    hidden = _sc_gather(ids, embedding_table).reshape(_B, _SEQ, _D)
    ext, bias = _tc_mask(encoder_attention_mask)
    ext = ext.reshape(_B, 1, 1, _SEQ)
    bias = bias.reshape(_B, _HEADS, _SEQ, 1)
    return (hidden, ext, bias)


# speedup vs baseline: 1.6821x; 1.0084x over previous
import functools

import jax
import jax.numpy as jnp
from jax import lax
from jax.experimental import pallas as pl
from jax.experimental.pallas import tpu as pltpu
from jax.experimental.pallas import tpu_sc as plsc

_B = 4
_SEQ = 4096
_D = 1024
_HEADS = 16

_NC = 2
_NS = 16
_NW = _NC * _NS
_N_IDS = _B * _SEQ
_PER_W = _N_IDS // _NW
_CHUNK = 32
_N_CHUNKS = _PER_W // _CHUNK


_NBUF = 3


def _gather_body(idx_hbm, table_hbm, out_hbm, idx_v, rows0, rows1, rows2,
                 gs0, gs1, gs2, os0, os1, os2):
    wid = lax.axis_index("s") * _NC + lax.axis_index("c")
    base = wid * _PER_W
    pltpu.sync_copy(idx_hbm.at[wid], idx_v)

    bufs = (rows0, rows1, rows2)
    gsems = (gs0, gs1, gs2)
    osems = (os0, os1, os2)

    def gather(i, b):
        pltpu.make_async_copy(table_hbm.at[idx_v.at[i]], bufs[b],
                              gsems[b]).start()

    def gather_wait(i, b):
        pltpu.make_async_copy(table_hbm.at[idx_v.at[i]], bufs[b],
                              gsems[b]).wait()

    def out_start(i, b):
        pltpu.make_async_copy(bufs[b],
                              out_hbm.at[pl.ds(base + i * _CHUNK, _CHUNK)],
                              osems[b]).start()

    def out_wait(i, b):
        pltpu.make_async_copy(bufs[b],
                              out_hbm.at[pl.ds(base + i * _CHUNK, _CHUNK)],
                              osems[b]).wait()

    for b in range(_NBUF):
        gather(b, b)

    def steady(j, _):
        for b in range(_NBUF):
            i = _NBUF * j + b
            gather_wait(i, b)
            out_start(i, b)
            out_wait(i, b)
            gather(i + _NBUF, b)
        return 0

    lax.fori_loop(0, _N_CHUNKS // _NBUF - 1, steady, 0)

    tail = _N_CHUNKS - _NBUF - (_N_CHUNKS % _NBUF)
    for i in range(tail, _N_CHUNKS - _NBUF):
        b = i % _NBUF
        gather_wait(i, b)
        out_start(i, b)
        out_wait(i, b)
        gather(i + _NBUF, b)
    for i in range(_N_CHUNKS - _NBUF, _N_CHUNKS):
        b = i % _NBUF
        gather_wait(i, b)
        out_start(i, b)
    for i in range(_N_CHUNKS - _NBUF, _N_CHUNKS):
        out_wait(i, i % _NBUF)


@jax.jit
def _sc_gather(ids_3d, table):
    mesh = plsc.VectorSubcoreMesh(core_axis_name="c", subcore_axis_name="s")
    f = functools.partial(
        pl.kernel,
        mesh=mesh,
        out_type=jax.ShapeDtypeStruct((_N_IDS, _D), jnp.float32),
        scratch_types=(
            [pltpu.VMEM((_N_CHUNKS, _CHUNK), jnp.int32)]
            + [pltpu.VMEM((_CHUNK, _D), jnp.float32)] * _NBUF
            + [pltpu.SemaphoreType.DMA] * (2 * _NBUF)
        ),
    )(_gather_body)
    return f(ids_3d, table)


def _mask_body(mask_ref, ext_ref, bias_ref):
    ext_ref[...] = (1.0 - mask_ref[...]) * -10000.0
    bias_ref[...] = jnp.zeros_like(bias_ref)


@jax.jit
def _tc_mask(mask):
    return pl.pallas_call(
        _mask_body,
        out_shape=[
            jax.ShapeDtypeStruct((_B, _SEQ), jnp.float32),
            jax.ShapeDtypeStruct((_B * _HEADS, _SEQ), jnp.float32),
        ],
    )(mask)


def kernel(encoder_input_ids, encoder_attention_mask, embedding_table):
    ids = encoder_input_ids.astype(jnp.int32).reshape(_NW, _N_CHUNKS, _CHUNK)
    hidden = _sc_gather(ids, embedding_table).reshape(_B, _SEQ, _D)
    ext, bias = _tc_mask(encoder_attention_mask)
    ext = ext.reshape(_B, 1, 1, _SEQ)
    bias = bias.reshape(_B, _HEADS, _SEQ, 1)
    return (hidden, ext, bias)
